# baseline (speedup 1.0000x reference)
import functools

import jax
import jax.numpy as jnp
import numpy as np
from jax import lax
from jax.experimental import pallas as pl
from jax.experimental.pallas import tpu as pltpu
from jax.experimental.pallas import tpu_sc as plsc

_N_EXPERTS = 8
_N_ACTIVE = 5
_TEMPERATURE = float(np.e)
_N_TOKENS = 2048
_BC = 512


def _scores_body(x_ref, gw_ref, gb_ref, s_ref):
    s = jnp.dot(x_ref[...], gw_ref[...], preferred_element_type=jnp.float32)
    s_ref[...] = (s + gb_ref[...]) / _TEMPERATURE


def _gate_scores(x, gate_W, gate_b):
    n, d = x.shape
    return pl.pallas_call(
        _scores_body,
        out_shape=jax.ShapeDtypeStruct((n, _N_EXPERTS), jnp.float32),
    )(x, gate_W, gate_b.reshape(1, -1))


def _sc_gate_weights(scores_t):
    nc, ns = 2, 16
    nw = nc * ns
    toks = _N_TOKENS // nw
    groups = toks // 16
    mesh = plsc.VectorSubcoreMesh(core_axis_name="c", subcore_axis_name="s",
                                  num_cores=nc)

    @functools.partial(
        pl.kernel, mesh=mesh,
        out_type=jax.ShapeDtypeStruct((_N_EXPERTS * _N_TOKENS,), jnp.float32),
        scratch_types=[pltpu.VMEM((_N_EXPERTS * toks,), jnp.float32),
                       pltpu.VMEM((_N_EXPERTS * toks,), jnp.float32)],
    )
    def k(s_hbm, w_hbm, sin, sout):
        wid = lax.axis_index("s") * nc + lax.axis_index("c")
        base = wid * toks
        for e in range(_N_EXPERTS):
            pltpu.sync_copy(s_hbm.at[pl.ds(e * _N_TOKENS + base, toks)],
                            sin.at[pl.ds(e * toks, toks)])
        for g in range(groups):
            s = [sin[pl.ds(e * toks + g * 16, 16)]
                 for e in range(_N_EXPERTS)]
            m = s[0]
            for e in range(1, _N_EXPERTS):
                m = jnp.maximum(m, s[e])
            p = [jnp.exp(s[e] - m) for e in range(_N_EXPERTS)]
            tot = p[0]
            for e in range(1, _N_EXPERTS):
                tot = tot + p[e]
            p = [p[e] / tot for e in range(_N_EXPERTS)]
            one = jnp.full((16,), 1.0, jnp.float32)
            zero = jnp.full((16,), 0.0, jnp.float32)
            sel = [zero for _ in range(_N_EXPERTS)]
            pw = list(p)
            for _ in range(_N_ACTIVE):
                mx = pw[0]
                for e in range(1, _N_EXPERTS):
                    mx = jnp.maximum(mx, pw[e])
                taken = zero
                for e in range(_N_EXPERTS):
                    eq = jnp.where(pw[e] == mx, one, zero)
                    take = eq * (one - taken)
                    taken = taken + take
                    sel[e] = sel[e] + take
                    pw[e] = pw[e] * (one - take) - take
            w = [p[e] * sel[e] for e in range(_N_EXPERTS)]
            norm = w[0]
            for e in range(1, _N_EXPERTS):
                norm = norm + w[e]
            norm = norm + 1e-8
            for e in range(_N_EXPERTS):
                sout[pl.ds(e * toks + g * 16, 16)] = w[e] / norm
        for e in range(_N_EXPERTS):
            pltpu.sync_copy(sout.at[pl.ds(e * toks, toks)],
                            w_hbm.at[pl.ds(e * _N_TOKENS + base, toks)])

    return k(scores_t.reshape(-1)).reshape(_N_EXPERTS, _N_TOKENS)


def _moe_body(x_ref, wts_ref, w1_ref, b1_ref, w2_ref, b2_ref,
              w3_ref, b3_ref, out_ref):
    e = pl.program_id(0)
    wfull = wts_ref[...]
    lane = jax.lax.broadcasted_iota(jnp.int32, wfull.shape, 1)
    w_all = jnp.sum(jnp.where(lane == e, wfull, 0.0), axis=-1,
                    keepdims=True)

    for c in range(_N_TOKENS // _BC):
        lo = c * _BC
        tok = slice(lo, lo + _BC)
        xb = x_ref[tok, :]
        h1 = jnp.dot(xb, w1_ref[0], preferred_element_type=jnp.float32)
        h1 = jnp.maximum(h1 + b1_ref[0], 0.0)
        h2 = jnp.dot(h1, w2_ref[0], preferred_element_type=jnp.float32)
        h2 = jnp.maximum(h2 + b2_ref[0], 0.0)
        o = jnp.dot(h2, w3_ref[0], preferred_element_type=jnp.float32)
        o = o + b3_ref[0]
        contrib = o * w_all[lo:lo + _BC, :]

        @pl.when(e == 0)
        def _init():
            out_ref[tok, :] = contrib

        @pl.when(e > 0)
        def _acc():
            out_ref[tok, :] += contrib


def _moe_mlp(x, wts, W1, b1, W2, b2, W3, b3):
    n, d = x.shape
    e, _, h = W1.shape
    o_dim = W3.shape[-1]
    return pl.pallas_call(
        _moe_body,
        grid=(e,),
        in_specs=[
            pl.BlockSpec((n, d), lambda ei: (0, 0)),
            pl.BlockSpec((n, _N_EXPERTS), lambda ei: (0, 0)),
            pl.BlockSpec((1, d, h), lambda ei: (ei, 0, 0)),
            pl.BlockSpec((1, 1, h), lambda ei: (ei, 0, 0)),
            pl.BlockSpec((1, h, h), lambda ei: (ei, 0, 0)),
            pl.BlockSpec((1, 1, h), lambda ei: (ei, 0, 0)),
            pl.BlockSpec((1, h, o_dim), lambda ei: (ei, 0, 0)),
            pl.BlockSpec((1, 1, o_dim), lambda ei: (ei, 0, 0)),
        ],
        out_specs=pl.BlockSpec((n, o_dim), lambda ei: (0, 0)),
        out_shape=jax.ShapeDtypeStruct((n, o_dim), jnp.float32),
        compiler_params=pltpu.CompilerParams(
            dimension_semantics=("arbitrary",),
            vmem_limit_bytes=100 * 1024 * 1024),
    )(x, wts, W1, b1.reshape(e, 1, h), W2, b2.reshape(e, 1, h),
      W3, b3.reshape(e, 1, o_dim))


@jax.jit
def kernel(x, gate_W, gate_b, W1, b1, W2, b2, W3, b3):
    scores = _gate_scores(x, gate_W, gate_b)
    wts = _sc_gate_weights(scores.T).T
    return _moe_mlp(x, wts, W1, b1, W2, b2, W3, b3)


# speedup vs baseline: 1.0187x; 1.0187x over previous
import functools

import jax
import jax.numpy as jnp
import numpy as np
from jax import lax
from jax.experimental import pallas as pl
from jax.experimental.pallas import tpu as pltpu
from jax.experimental.pallas import tpu_sc as plsc

_N_EXPERTS = 8
_N_ACTIVE = 5
_TEMPERATURE = float(np.e)
_N_TOKENS = 2048
_BC = 512


def _scores_body(x_ref, gw_ref, gb_ref, s_ref):
    s = jnp.dot(x_ref[...], gw_ref[...], preferred_element_type=jnp.float32)
    s_ref[...] = (s + gb_ref[...]) / _TEMPERATURE


def _gate_scores(x, gate_W, gate_b):
    n, d = x.shape
    return pl.pallas_call(
        _scores_body,
        out_shape=jax.ShapeDtypeStruct((n, _N_EXPERTS), jnp.float32),
    )(x, gate_W, gate_b.reshape(1, -1))


def _sc_gate_weights(scores_t):
    nc, ns = 2, 16
    nw = nc * ns
    toks = _N_TOKENS // nw
    groups = toks // 16
    mesh = plsc.VectorSubcoreMesh(core_axis_name="c", subcore_axis_name="s",
                                  num_cores=nc)

    @functools.partial(
        pl.kernel, mesh=mesh,
        out_type=jax.ShapeDtypeStruct((_N_EXPERTS * _N_TOKENS,), jnp.float32),
        scratch_types=[pltpu.VMEM((_N_EXPERTS * toks,), jnp.float32),
                       pltpu.VMEM((_N_EXPERTS * toks,), jnp.float32),
                       pltpu.SemaphoreType.DMA,
                       pltpu.SemaphoreType.DMA],
    )
    def k(s_hbm, w_hbm, sin, sout, sem_in, sem_out):
        wid = lax.axis_index("s") * nc + lax.axis_index("c")
        base = wid * toks
        copies = [pltpu.async_copy(
            s_hbm.at[pl.ds(e * _N_TOKENS + base, toks)],
            sin.at[pl.ds(e * toks, toks)], sem_in)
            for e in range(_N_EXPERTS)]
        for c in copies:
            c.wait()
        for g in range(groups):
            s = [sin[pl.ds(e * toks + g * 16, 16)]
                 for e in range(_N_EXPERTS)]
            m = s[0]
            for e in range(1, _N_EXPERTS):
                m = jnp.maximum(m, s[e])
            p = [jnp.exp(s[e] - m) for e in range(_N_EXPERTS)]
            tot = p[0]
            for e in range(1, _N_EXPERTS):
                tot = tot + p[e]
            p = [p[e] / tot for e in range(_N_EXPERTS)]
            one = jnp.full((16,), 1.0, jnp.float32)
            zero = jnp.full((16,), 0.0, jnp.float32)
            sel = [zero for _ in range(_N_EXPERTS)]
            pw = list(p)
            for _ in range(_N_ACTIVE):
                mx = pw[0]
                for e in range(1, _N_EXPERTS):
                    mx = jnp.maximum(mx, pw[e])
                taken = zero
                for e in range(_N_EXPERTS):
                    eq = jnp.where(pw[e] == mx, one, zero)
                    take = eq * (one - taken)
                    taken = taken + take
                    sel[e] = sel[e] + take
                    pw[e] = pw[e] * (one - take) - take
            w = [p[e] * sel[e] for e in range(_N_EXPERTS)]
            norm = w[0]
            for e in range(1, _N_EXPERTS):
                norm = norm + w[e]
            norm = norm + 1e-8
            for e in range(_N_EXPERTS):
                sout[pl.ds(e * toks + g * 16, 16)] = w[e] / norm
        out_copies = [pltpu.async_copy(
            sout.at[pl.ds(e * toks, toks)],
            w_hbm.at[pl.ds(e * _N_TOKENS + base, toks)], sem_out)
            for e in range(_N_EXPERTS)]
        for c in out_copies:
            c.wait()

    return k(scores_t.reshape(-1)).reshape(_N_EXPERTS, _N_TOKENS)


def _moe_body(x_ref, wts_ref, w1_ref, b1_ref, w2_ref, b2_ref,
              w3_ref, b3_ref, out_ref):
    e = pl.program_id(0)
    wfull = wts_ref[...]
    lane = jax.lax.broadcasted_iota(jnp.int32, wfull.shape, 1)
    w_all = jnp.sum(jnp.where(lane == e, wfull, 0.0), axis=-1,
                    keepdims=True)

    for c in range(_N_TOKENS // _BC):
        lo = c * _BC
        tok = slice(lo, lo + _BC)
        xb = x_ref[tok, :]
        h1 = jnp.dot(xb, w1_ref[0], preferred_element_type=jnp.float32)
        h1 = jnp.maximum(h1 + b1_ref[0], 0.0)
        h2 = jnp.dot(h1, w2_ref[0], preferred_element_type=jnp.float32)
        h2 = jnp.maximum(h2 + b2_ref[0], 0.0)
        o = jnp.dot(h2, w3_ref[0], preferred_element_type=jnp.float32)
        o = o + b3_ref[0]
        contrib = o * w_all[lo:lo + _BC, :]

        @pl.when(e == 0)
        def _init():
            out_ref[tok, :] = contrib

        @pl.when(e > 0)
        def _acc():
            out_ref[tok, :] += contrib


def _moe_mlp(x, wts, W1, b1, W2, b2, W3, b3):
    n, d = x.shape
    e, _, h = W1.shape
    o_dim = W3.shape[-1]
    return pl.pallas_call(
        _moe_body,
        grid=(e,),
        in_specs=[
            pl.BlockSpec((n, d), lambda ei: (0, 0)),
            pl.BlockSpec((n, _N_EXPERTS), lambda ei: (0, 0)),
            pl.BlockSpec((1, d, h), lambda ei: (ei, 0, 0)),
            pl.BlockSpec((1, 1, h), lambda ei: (ei, 0, 0)),
            pl.BlockSpec((1, h, h), lambda ei: (ei, 0, 0)),
            pl.BlockSpec((1, 1, h), lambda ei: (ei, 0, 0)),
            pl.BlockSpec((1, h, o_dim), lambda ei: (ei, 0, 0)),
            pl.BlockSpec((1, 1, o_dim), lambda ei: (ei, 0, 0)),
        ],
        out_specs=pl.BlockSpec((n, o_dim), lambda ei: (0, 0)),
        out_shape=jax.ShapeDtypeStruct((n, o_dim), jnp.float32),
        compiler_params=pltpu.CompilerParams(
            dimension_semantics=("arbitrary",),
            vmem_limit_bytes=100 * 1024 * 1024),
    )(x, wts, W1, b1.reshape(e, 1, h), W2, b2.reshape(e, 1, h),
      W3, b3.reshape(e, 1, o_dim))


@jax.jit
def kernel(x, gate_W, gate_b, W1, b1, W2, b2, W3, b3):
    scores = _gate_scores(x, gate_W, gate_b)
    wts = _sc_gate_weights(scores.T).T
    return _moe_mlp(x, wts, W1, b1, W2, b2, W3, b3)


# speedup vs baseline: 1.0316x; 1.0126x over previous
import functools

import jax
import jax.numpy as jnp
import numpy as np
from jax import lax
from jax.experimental import pallas as pl
from jax.experimental.pallas import tpu as pltpu
from jax.experimental.pallas import tpu_sc as plsc

_N_EXPERTS = 8
_N_ACTIVE = 5
_TEMPERATURE = float(np.e)
_N_TOKENS = 2048
_BC = 512


def _scores_body(x_ref, gw_ref, gb_ref, s_ref):
    s = jnp.dot(x_ref[...], gw_ref[...], preferred_element_type=jnp.float32)
    s = (s + gb_ref[...]) / _TEMPERATURE
    s_ref[...] = s.T


def _gate_scores(x, gate_W, gate_b):
    n, d = x.shape
    return pl.pallas_call(
        _scores_body,
        out_shape=jax.ShapeDtypeStruct((_N_EXPERTS, n), jnp.float32),
    )(x, gate_W, gate_b.reshape(1, -1))


def _sc_gate_weights(scores_t):
    nc, ns = 2, 16
    nw = nc * ns
    toks = _N_TOKENS // nw
    groups = toks // 16
    mesh = plsc.VectorSubcoreMesh(core_axis_name="c", subcore_axis_name="s",
                                  num_cores=nc)

    @functools.partial(
        pl.kernel, mesh=mesh,
        out_type=jax.ShapeDtypeStruct((_N_EXPERTS * _N_TOKENS,), jnp.float32),
        scratch_types=[pltpu.VMEM((_N_EXPERTS * toks,), jnp.float32),
                       pltpu.VMEM((_N_EXPERTS * toks,), jnp.float32),
                       pltpu.SemaphoreType.DMA,
                       pltpu.SemaphoreType.DMA],
    )
    def k(s_hbm, w_hbm, sin, sout, sem_in, sem_out):
        wid = lax.axis_index("s") * nc + lax.axis_index("c")
        base = wid * toks
        copies = [pltpu.async_copy(
            s_hbm.at[pl.ds(e * _N_TOKENS + base, toks)],
            sin.at[pl.ds(e * toks, toks)], sem_in)
            for e in range(_N_EXPERTS)]
        for c in copies:
            c.wait()
        for g in range(groups):
            s = [sin[pl.ds(e * toks + g * 16, 16)]
                 for e in range(_N_EXPERTS)]
            m = s[0]
            for e in range(1, _N_EXPERTS):
                m = jnp.maximum(m, s[e])
            p = [jnp.exp(s[e] - m) for e in range(_N_EXPERTS)]
            tot = p[0]
            for e in range(1, _N_EXPERTS):
                tot = tot + p[e]
            p = [p[e] / tot for e in range(_N_EXPERTS)]
            one = jnp.full((16,), 1.0, jnp.float32)
            zero = jnp.full((16,), 0.0, jnp.float32)
            sel = [zero for _ in range(_N_EXPERTS)]
            pw = list(p)
            for _ in range(_N_ACTIVE):
                mx = pw[0]
                for e in range(1, _N_EXPERTS):
                    mx = jnp.maximum(mx, pw[e])
                taken = zero
                for e in range(_N_EXPERTS):
                    eq = jnp.where(pw[e] == mx, one, zero)
                    take = eq * (one - taken)
                    taken = taken + take
                    sel[e] = sel[e] + take
                    pw[e] = pw[e] * (one - take) - take
            w = [p[e] * sel[e] for e in range(_N_EXPERTS)]
            norm = w[0]
            for e in range(1, _N_EXPERTS):
                norm = norm + w[e]
            norm = norm + 1e-8
            for e in range(_N_EXPERTS):
                sout[pl.ds(e * toks + g * 16, 16)] = w[e] / norm
        out_copies = [pltpu.async_copy(
            sout.at[pl.ds(e * toks, toks)],
            w_hbm.at[pl.ds(e * _N_TOKENS + base, toks)], sem_out)
            for e in range(_N_EXPERTS)]
        for c in out_copies:
            c.wait()

    return k(scores_t.reshape(-1)).reshape(_N_EXPERTS, _N_TOKENS)


def _moe_body(x_ref, wts_t_ref, w1_ref, b1_ref, w2_ref, b2_ref,
              w3_ref, b3_ref, out_ref, wts_ref):
    e = pl.program_id(0)

    @pl.when(e == 0)
    def _detranspose():
        wts_ref[...] = wts_t_ref[...].T

    wfull = wts_ref[...]
    lane = jax.lax.broadcasted_iota(jnp.int32, wfull.shape, 1)
    w_all = jnp.sum(jnp.where(lane == e, wfull, 0.0), axis=-1,
                    keepdims=True)

    for c in range(_N_TOKENS // _BC):
        lo = c * _BC
        tok = slice(lo, lo + _BC)
        xb = x_ref[tok, :]
        h1 = jnp.dot(xb, w1_ref[0], preferred_element_type=jnp.float32)
        h1 = jnp.maximum(h1 + b1_ref[0], 0.0)
        h2 = jnp.dot(h1, w2_ref[0], preferred_element_type=jnp.float32)
        h2 = jnp.maximum(h2 + b2_ref[0], 0.0)
        o = jnp.dot(h2, w3_ref[0], preferred_element_type=jnp.float32)
        o = o + b3_ref[0]
        contrib = o * w_all[lo:lo + _BC, :]

        @pl.when(e == 0)
        def _init():
            out_ref[tok, :] = contrib

        @pl.when(e > 0)
        def _acc():
            out_ref[tok, :] += contrib


def _moe_mlp(x, wts_t, W1, b1, W2, b2, W3, b3):
    n, d = x.shape
    e, _, h = W1.shape
    o_dim = W3.shape[-1]
    return pl.pallas_call(
        _moe_body,
        grid=(e,),
        in_specs=[
            pl.BlockSpec((n, d), lambda ei: (0, 0)),
            pl.BlockSpec((_N_EXPERTS, n), lambda ei: (0, 0)),
            pl.BlockSpec((1, d, h), lambda ei: (ei, 0, 0)),
            pl.BlockSpec((1, 1, h), lambda ei: (ei, 0, 0)),
            pl.BlockSpec((1, h, h), lambda ei: (ei, 0, 0)),
            pl.BlockSpec((1, 1, h), lambda ei: (ei, 0, 0)),
            pl.BlockSpec((1, h, o_dim), lambda ei: (ei, 0, 0)),
            pl.BlockSpec((1, 1, o_dim), lambda ei: (ei, 0, 0)),
        ],
        out_specs=pl.BlockSpec((n, o_dim), lambda ei: (0, 0)),
        out_shape=jax.ShapeDtypeStruct((n, o_dim), jnp.float32),
        scratch_shapes=[pltpu.VMEM((n, _N_EXPERTS), jnp.float32)],
        compiler_params=pltpu.CompilerParams(
            dimension_semantics=("arbitrary",),
            vmem_limit_bytes=100 * 1024 * 1024),
    )(x, wts_t, W1, b1.reshape(e, 1, h), W2, b2.reshape(e, 1, h),
      W3, b3.reshape(e, 1, o_dim))


@jax.jit
def kernel(x, gate_W, gate_b, W1, b1, W2, b2, W3, b3):
    scores_t = _gate_scores(x, gate_W, gate_b)
    wts_t = _sc_gate_weights(scores_t)
    return _moe_mlp(x, wts_t, W1, b1, W2, b2, W3, b3)


# speedup vs baseline: 1.0512x; 1.0190x over previous
import functools

import jax
import jax.numpy as jnp
import numpy as np
from jax import lax
from jax.experimental import pallas as pl
from jax.experimental.pallas import tpu as pltpu
from jax.experimental.pallas import tpu_sc as plsc

_N_EXPERTS = 8
_N_ACTIVE = 5
_TEMPERATURE = float(np.e)
_N_TOKENS = 2048
_BC = 1024


def _scores_body(x_ref, gw_ref, gb_ref, s_ref):
    s = jnp.dot(x_ref[...], gw_ref[...], preferred_element_type=jnp.float32)
    s = (s + gb_ref[...]) / _TEMPERATURE
    s_ref[...] = s.T


def _gate_scores(x, gate_W, gate_b):
    n, d = x.shape
    return pl.pallas_call(
        _scores_body,
        out_shape=jax.ShapeDtypeStruct((_N_EXPERTS, n), jnp.float32),
    )(x, gate_W, gate_b.reshape(1, -1))


def _sc_gate_weights(scores_t):
    nc, ns = 2, 16
    nw = nc * ns
    toks = _N_TOKENS // nw
    groups = toks // 16
    mesh = plsc.VectorSubcoreMesh(core_axis_name="c", subcore_axis_name="s",
                                  num_cores=nc)

    @functools.partial(
        pl.kernel, mesh=mesh,
        out_type=jax.ShapeDtypeStruct((_N_EXPERTS * _N_TOKENS,), jnp.float32),
        scratch_types=[pltpu.VMEM((_N_EXPERTS * toks,), jnp.float32),
                       pltpu.VMEM((_N_EXPERTS * toks,), jnp.float32),
                       pltpu.SemaphoreType.DMA,
                       pltpu.SemaphoreType.DMA],
    )
    def k(s_hbm, w_hbm, sin, sout, sem_in, sem_out):
        wid = lax.axis_index("s") * nc + lax.axis_index("c")
        base = wid * toks
        copies = [pltpu.async_copy(
            s_hbm.at[pl.ds(e * _N_TOKENS + base, toks)],
            sin.at[pl.ds(e * toks, toks)], sem_in)
            for e in range(_N_EXPERTS)]
        for c in copies:
            c.wait()
        for g in range(groups):
            s = [sin[pl.ds(e * toks + g * 16, 16)]
                 for e in range(_N_EXPERTS)]
            m = s[0]
            for e in range(1, _N_EXPERTS):
                m = jnp.maximum(m, s[e])
            p = [jnp.exp(s[e] - m) for e in range(_N_EXPERTS)]
            tot = p[0]
            for e in range(1, _N_EXPERTS):
                tot = tot + p[e]
            p = [p[e] / tot for e in range(_N_EXPERTS)]
            one = jnp.full((16,), 1.0, jnp.float32)
            zero = jnp.full((16,), 0.0, jnp.float32)
            sel = [zero for _ in range(_N_EXPERTS)]
            pw = list(p)
            for _ in range(_N_ACTIVE):
                mx = pw[0]
                for e in range(1, _N_EXPERTS):
                    mx = jnp.maximum(mx, pw[e])
                taken = zero
                for e in range(_N_EXPERTS):
                    eq = jnp.where(pw[e] == mx, one, zero)
                    take = eq * (one - taken)
                    taken = taken + take
                    sel[e] = sel[e] + take
                    pw[e] = pw[e] * (one - take) - take
            w = [p[e] * sel[e] for e in range(_N_EXPERTS)]
            norm = w[0]
            for e in range(1, _N_EXPERTS):
                norm = norm + w[e]
            norm = norm + 1e-8
            for e in range(_N_EXPERTS):
                sout[pl.ds(e * toks + g * 16, 16)] = w[e] / norm
        out_copies = [pltpu.async_copy(
            sout.at[pl.ds(e * toks, toks)],
            w_hbm.at[pl.ds(e * _N_TOKENS + base, toks)], sem_out)
            for e in range(_N_EXPERTS)]
        for c in out_copies:
            c.wait()

    return k(scores_t.reshape(-1)).reshape(_N_EXPERTS, _N_TOKENS)


def _moe_body(x_ref, wts_t_ref, w1_ref, b1_ref, w2_ref, b2_ref,
              w3_ref, b3_ref, out_ref, wts_ref):
    e = pl.program_id(0)

    @pl.when(e == 0)
    def _detranspose():
        wts_ref[...] = wts_t_ref[...].T

    wfull = wts_ref[...]
    lane = jax.lax.broadcasted_iota(jnp.int32, wfull.shape, 1)
    w_all = jnp.sum(jnp.where(lane == e, wfull, 0.0), axis=-1,
                    keepdims=True)

    for c in range(_N_TOKENS // _BC):
        lo = c * _BC
        tok = slice(lo, lo + _BC)
        xb = x_ref[tok, :]
        h1 = jnp.dot(xb, w1_ref[0], preferred_element_type=jnp.float32)
        h1 = jnp.maximum(h1 + b1_ref[0], 0.0)
        h2 = jnp.dot(h1, w2_ref[0], preferred_element_type=jnp.float32)
        h2 = jnp.maximum(h2 + b2_ref[0], 0.0)
        o = jnp.dot(h2, w3_ref[0], preferred_element_type=jnp.float32)
        o = o + b3_ref[0]
        contrib = o * w_all[lo:lo + _BC, :]

        @pl.when(e == 0)
        def _init():
            out_ref[tok, :] = contrib

        @pl.when(e > 0)
        def _acc():
            out_ref[tok, :] += contrib


def _moe_mlp(x, wts_t, W1, b1, W2, b2, W3, b3):
    n, d = x.shape
    e, _, h = W1.shape
    o_dim = W3.shape[-1]
    return pl.pallas_call(
        _moe_body,
        grid=(e,),
        in_specs=[
            pl.BlockSpec((n, d), lambda ei: (0, 0)),
            pl.BlockSpec((_N_EXPERTS, n), lambda ei: (0, 0)),
            pl.BlockSpec((1, d, h), lambda ei: (ei, 0, 0)),
            pl.BlockSpec((1, 1, h), lambda ei: (ei, 0, 0)),
            pl.BlockSpec((1, h, h), lambda ei: (ei, 0, 0)),
            pl.BlockSpec((1, 1, h), lambda ei: (ei, 0, 0)),
            pl.BlockSpec((1, h, o_dim), lambda ei: (ei, 0, 0)),
            pl.BlockSpec((1, 1, o_dim), lambda ei: (ei, 0, 0)),
        ],
        out_specs=pl.BlockSpec((n, o_dim), lambda ei: (0, 0)),
        out_shape=jax.ShapeDtypeStruct((n, o_dim), jnp.float32),
        scratch_shapes=[pltpu.VMEM((n, _N_EXPERTS), jnp.float32)],
        compiler_params=pltpu.CompilerParams(
            dimension_semantics=("arbitrary",),
            vmem_limit_bytes=100 * 1024 * 1024),
    )(x, wts_t, W1, b1.reshape(e, 1, h), W2, b2.reshape(e, 1, h),
      W3, b3.reshape(e, 1, o_dim))


@jax.jit
def kernel(x, gate_W, gate_b, W1, b1, W2, b2, W3, b3):
    scores_t = _gate_scores(x, gate_W, gate_b)
    wts_t = _sc_gate_weights(scores_t)
    return _moe_mlp(x, wts_t, W1, b1, W2, b2, W3, b3)
